# baseline (speedup 1.0000x reference)
import functools

import jax
import jax.numpy as jnp
from jax import lax
from jax.experimental import pallas as pl
from jax.experimental.pallas import tpu as pltpu
from jax.experimental.pallas import tpu_sc as plsc

_NC = 2
_NS = 16
_NW = _NC * _NS
_G = 128
_U = 4
_EPS = 1e-12


def _rsqrt_vec(v):
    bits = lax.bitcast_convert_type(v, jnp.int32)
    y = lax.bitcast_convert_type(
        jnp.int32(0x5F3759DF) - lax.shift_right_logical(bits, 1), jnp.float32)
    for _ in range(2):
        y = y * (1.5 - 0.5 * v * y * y)
    return y


def _make_fused(n, d, l):
    per_w = n // _NW
    n_chunks = per_w // _G
    d2 = 2 * d
    nk = d // 16
    mesh = plsc.VectorSubcoreMesh(core_axis_name="c", subcore_axis_name="s")

    @functools.partial(
        pl.kernel,
        mesh=mesh,
        out_type=jax.ShapeDtypeStruct((n, d), jnp.float32),
        scratch_types=[
            pltpu.VMEM((n_chunks, _G), jnp.int32),
            pltpu.VMEM((2, _G, d2), jnp.float32),
            pltpu.VMEM((2, _G, d), jnp.float32),
            pltpu.VMEM((l, d2), jnp.float32),
            pltpu.VMEM((8, d2), jnp.float32),
            pltpu.VMEM((8, d2), jnp.float32),
            pltpu.SemaphoreType.DMA,
            pltpu.SemaphoreType.DMA,
        ],
        compiler_params=pltpu.CompilerParams(needs_layout_passes=False),
    )
    def fused_k(idx_hbm, tab_hbm, pe_hbm, g_hbm, b_hbm, out_hbm,
                idx_v, rows_v, pack_v, pe_v, g_v, b_v, gsem, osem):
        wid = lax.axis_index("s") * _NC + lax.axis_index("c")
        base = wid * per_w
        pltpu.sync_copy(
            idx_hbm.at[pl.ds(pl.multiple_of(wid * n_chunks, 8), n_chunks)],
            idx_v)
        pltpu.sync_copy(pe_hbm, pe_v)
        pltpu.sync_copy(g_hbm, g_v)
        pltpu.sync_copy(b_hbm, b_v)
        gs = [g_v[0, pl.ds(k * 16, 16)] for k in range(nk)]
        bs = [b_v[0, pl.ds(k * 16, 16)] for k in range(nk)]

        pltpu.async_copy(tab_hbm.at[idx_v.at[0]], rows_v.at[0], gsem)

        def body(i, carry):
            sl = i % 2
            r0 = pl.multiple_of(base + i * _G, 128)
            pltpu.make_async_copy(
                tab_hbm.at[idx_v.at[i]], rows_v.at[sl], gsem).wait()

            @pl.when(i + 1 < n_chunks)
            def _prefetch():
                pltpu.async_copy(
                    tab_hbm.at[idx_v.at[i + 1]], rows_v.at[1 - sl], gsem)

            @pl.when(i >= 2)
            def _drain():
                pltpu.make_async_copy(
                    pack_v.at[sl], out_hbm.at[pl.ds(0, _G)], osem).wait()

            pm = lax.rem(r0, l)

            def rows(jj, c2):
                j0 = jj * _U
                for u in range(_U):
                    j = j0 + u
                    p0 = pm + j
                    p = jnp.where(p0 >= l, p0 - l, p0)
                    xs = [rows_v[sl, j, pl.ds(k * 16, 16)]
                          + pe_v[p, pl.ds(k * 16, 16)] for k in range(nk)]
                    s = (xs[0] + xs[1]) + (xs[2] + xs[3])
                    q = (xs[0] * xs[0] + xs[1] * xs[1]
                         + xs[2] * xs[2] + xs[3] * xs[3])
                    total = jnp.sum(s)
                    sumsq = jnp.sum(q)
                    mean = total * (1.0 / d)
                    var = sumsq * (1.0 / d) - mean * mean
                    vv = jnp.full((16,), var + _EPS, dtype=jnp.float32)
                    rstd = _rsqrt_vec(vv)
                    for k in range(nk):
                        pack_v[sl, j, pl.ds(k * 16, 16)] = (
                            (xs[k] - mean) * rstd * gs[k] + bs[k])
                return c2

            lax.fori_loop(0, _G // _U, rows, 0)
            pltpu.async_copy(pack_v.at[sl], out_hbm.at[pl.ds(r0, _G)], osem)
            return carry

        lax.fori_loop(0, n_chunks, body, 0)
        pltpu.make_async_copy(
            pack_v.at[0], out_hbm.at[pl.ds(0, _G)], osem).wait()
        pltpu.make_async_copy(
            pack_v.at[1], out_hbm.at[pl.ds(0, _G)], osem).wait()

    return fused_k


def kernel(input_ids, W_emb, pe, ln_gamma, ln_beta):
    b, l = input_ids.shape
    v, d = W_emb.shape
    n = b * l
    idx = input_ids.reshape(n // _G, _G)
    tab = jnp.pad(W_emb, ((0, 0), (0, d)))
    pe_pad = jnp.pad(pe[:l], ((0, 0), (0, d)))
    g_pad = jnp.pad(ln_gamma.reshape(1, d), ((0, 7), (0, d)))
    b_pad = jnp.pad(ln_beta.reshape(1, d), ((0, 7), (0, d)))
    fused = _make_fused(n, d, l)
    out = fused(idx, tab, pe_pad, g_pad, b_pad)
    return out.reshape(b, l, d)


# speedup vs baseline: 1.3049x; 1.3049x over previous
import functools

import jax
import jax.numpy as jnp
from jax import lax
from jax.experimental import pallas as pl
from jax.experimental.pallas import tpu as pltpu
from jax.experimental.pallas import tpu_sc as plsc

_NC = 2
_NS = 16
_NW = _NC * _NS
_G = 128
_EPS = 1e-12


def _make_gather(n_rows, d, chj):
    per_w = n_rows // _NW
    n_chunks = per_w // chj
    mesh = plsc.VectorSubcoreMesh(core_axis_name="c", subcore_axis_name="s")

    @functools.partial(
        pl.kernel,
        mesh=mesh,
        out_type=jax.ShapeDtypeStruct((n_rows, _G, d), jnp.float32),
        scratch_types=[
            pltpu.VMEM((chj, _G), jnp.int32),
            pltpu.VMEM((chj, _G, d), jnp.float32),
            pltpu.SemaphoreType.DMA,
        ],
        compiler_params=pltpu.CompilerParams(use_tc_tiling_on_sc=False),
    )
    def gather_k(idx_hbm, table_hbm, out_hbm, idx_v, rows_v, sem):
        wid = lax.axis_index("s") * _NC + lax.axis_index("c")
        base = wid * per_w

        def body(i, carry):
            r0 = base + i * chj
            pltpu.sync_copy(idx_hbm.at[pl.ds(r0, chj)], idx_v)
            handles = [
                pltpu.async_copy(table_hbm.at[idx_v.at[j]], rows_v.at[j], sem)
                for j in range(chj)
            ]
            for h in handles:
                h.wait()
            pltpu.sync_copy(rows_v, out_hbm.at[pl.ds(r0, chj)])
            return carry

        lax.fori_loop(0, n_chunks, body, 0)

    return gather_k


def _ln_pairs_body(x_ref, pe_ref, g_ref, bta_ref, o_ref):
    x = x_ref[...] + pe_ref[...]
    col = lax.broadcasted_iota(jnp.int32, (128, 128), 0) // 64
    row = lax.broadcasted_iota(jnp.int32, (128, 128), 1) // 64
    m = (col == row).astype(jnp.float32)
    mean = jnp.dot(x, m, preferred_element_type=jnp.float32) * (1.0 / 64.0)
    c = x - mean
    var = jnp.dot(c * c, m, preferred_element_type=jnp.float32) * (1.0 / 64.0)
    o_ref[...] = c * lax.rsqrt(var + _EPS) * g_ref[...] + bta_ref[...]


def _make_ln_pairs(rb, rows):
    return pl.pallas_call(
        _ln_pairs_body,
        grid=(rows // rb,),
        in_specs=[
            pl.BlockSpec((rb, 128), lambda i: (i, 0)),
            pl.BlockSpec((rb, 128), lambda i: (0, 0)),
            pl.BlockSpec((1, 128), lambda i: (0, 0)),
            pl.BlockSpec((1, 128), lambda i: (0, 0)),
        ],
        out_specs=pl.BlockSpec((rb, 128), lambda i: (i, 0)),
        out_shape=jax.ShapeDtypeStruct((rows, 128), jnp.float32),
    )


def kernel(input_ids, W_emb, pe, ln_gamma, ln_beta):
    b, l = input_ids.shape
    d = W_emb.shape[1]
    n = b * l
    n_rows = n // _G
    idx = input_ids.reshape(n_rows, _G)
    gathered = _make_gather(n_rows, d, 4)(idx, W_emb)
    rows = n // 2
    x = gathered.reshape(rows, 2 * d)
    rb = 3200
    pe_pairs = jnp.tile(pe[:l].reshape(l // 2, 2 * d), (rb // (l // 2), 1))
    g2 = jnp.concatenate([ln_gamma, ln_gamma]).reshape(1, 2 * d)
    b2 = jnp.concatenate([ln_beta, ln_beta]).reshape(1, 2 * d)
    out = _make_ln_pairs(rb, rows)(x, pe_pairs, g2, b2)
    return out.reshape(b, l, d)


# speedup vs baseline: 1.3377x; 1.0251x over previous
import functools

import jax
import jax.numpy as jnp
from jax import lax
from jax.experimental import pallas as pl
from jax.experimental.pallas import tpu as pltpu
from jax.experimental.pallas import tpu_sc as plsc

_NC = 2
_NS = 16
_NW = _NC * _NS
_G = 128
_EPS = 1e-12


def _make_gather(n_rows, d, chj):
    per_w = n_rows // _NW
    n_chunks = per_w // chj
    mesh = plsc.VectorSubcoreMesh(core_axis_name="c", subcore_axis_name="s")

    @functools.partial(
        pl.kernel,
        mesh=mesh,
        out_type=jax.ShapeDtypeStruct((n_rows, _G, d), jnp.float32),
        scratch_types=[
            pltpu.VMEM((chj, _G), jnp.int32),
            pltpu.VMEM((chj, _G, d), jnp.float32),
            pltpu.SemaphoreType.DMA,
        ],
        compiler_params=pltpu.CompilerParams(use_tc_tiling_on_sc=False),
    )
    def gather_k(idx_hbm, table_hbm, out_hbm, idx_v, rows_v, sem):
        wid = lax.axis_index("s") * _NC + lax.axis_index("c")
        base = wid * per_w

        def body(i, carry):
            r0 = base + i * chj
            pltpu.sync_copy(idx_hbm.at[pl.ds(r0, chj)], idx_v)
            handles = [
                pltpu.async_copy(table_hbm.at[idx_v.at[j]], rows_v.at[j], sem)
                for j in range(chj)
            ]
            for h in handles:
                h.wait()
            pltpu.sync_copy(rows_v, out_hbm.at[pl.ds(r0, chj)])
            return carry

        lax.fori_loop(0, n_chunks, body, 0)

    return gather_k


def _ln_pairs_body(x_ref, pe_ref, g_ref, bta_ref, o_ref):
    x = x_ref[...] + pe_ref[...]
    col = lax.broadcasted_iota(jnp.int32, (128, 128), 0) // 64
    row = lax.broadcasted_iota(jnp.int32, (128, 128), 1) // 64
    m = (col == row).astype(jnp.float32)
    mean = jnp.dot(x, m, preferred_element_type=jnp.float32) * (1.0 / 64.0)
    c = x - mean
    var = jnp.dot(c * c, m, preferred_element_type=jnp.float32) * (1.0 / 64.0)
    o_ref[...] = c * lax.rsqrt(var + _EPS) * g_ref[...] + bta_ref[...]


def _make_ln_pairs(rb, rows):
    return pl.pallas_call(
        _ln_pairs_body,
        grid=(rows // rb,),
        in_specs=[
            pl.BlockSpec((rb, 128), lambda i: (i, 0)),
            pl.BlockSpec((rb, 128), lambda i: (0, 0)),
            pl.BlockSpec((1, 128), lambda i: (0, 0)),
            pl.BlockSpec((1, 128), lambda i: (0, 0)),
        ],
        out_specs=pl.BlockSpec((rb, 128), lambda i: (i, 0)),
        out_shape=jax.ShapeDtypeStruct((rows, 128), jnp.float32),
    )


def kernel(input_ids, W_emb, pe, ln_gamma, ln_beta):
    b, l = input_ids.shape
    d = W_emb.shape[1]
    n = b * l
    n_rows = n // _G
    idx = input_ids.reshape(n_rows, _G)
    gathered = _make_gather(n_rows, d, 4)(idx, W_emb)
    rows = n // 2
    x = gathered.reshape(rows, 2 * d)
    rb = 12800
    pe_pairs = jnp.tile(pe[:l].reshape(l // 2, 2 * d), (rb // (l // 2), 1))
    g2 = jnp.concatenate([ln_gamma, ln_gamma]).reshape(1, 2 * d)
    b2 = jnp.concatenate([ln_beta, ln_beta]).reshape(1, 2 * d)
    out = _make_ln_pairs(rb, rows)(x, pe_pairs, g2, b2)
    return out.reshape(b, l, d)
